# baseline (speedup 1.0000x reference)
import functools
import jax
import jax.numpy as jnp
from jax import lax
from jax.experimental import pallas as pl
from jax.experimental.pallas import tpu as pltpu
from jax.experimental.pallas import tpu_sc as plsc

_NC = 2
_NS = 16
_NW = _NC * _NS
_LANES = 16


def _make_sc_add(B, L, D):
    ROWS = 16
    l_per_w = L // _NW
    n_sub = l_per_w // ROWS
    N = n_sub * B
    GROUPS = D // _LANES

    mesh = plsc.VectorSubcoreMesh(core_axis_name="c", subcore_axis_name="s")

    @functools.partial(
        pl.kernel,
        mesh=mesh,
        out_type=jax.ShapeDtypeStruct((B, L, D), jnp.float32),
        scratch_types=(
            [pltpu.VMEM((ROWS, D), jnp.float32) for _ in range(4)]
            + [pltpu.VMEM((ROWS, D), jnp.float32) for _ in range(2)]
            + [pltpu.SemaphoreType.DMA for _ in range(4)]
            + [pltpu.SemaphoreType.DMA for _ in range(4)]
            + [pltpu.SemaphoreType.DMA for _ in range(2)]
        ),
    )
    def k(x_hbm, pe_hbm, o_hbm, *refs):
        xb = refs[0:4]
        peb = refs[4:6]
        lsem = refs[6:10]
        ssem = refs[10:14]
        psem = refs[14:16]

        wid = lax.axis_index("s") * _NC + lax.axis_index("c")
        base_l = wid * l_per_w

        def l0(c):
            return base_l + c * ROWS

        pe_pend = {}
        for c in range(min(2, n_sub)):
            pe_pend[c] = pltpu.async_copy(
                pe_hbm.at[pl.ds(l0(c), ROWS)], peb[c % 2], psem[c % 2])
        ld = {}
        for g in range(min(3, N)):
            c, b = g // B, g % B
            ld[g] = pltpu.async_copy(
                x_hbm.at[b, pl.ds(l0(c), ROWS)], xb[g % 4], lsem[g % 4])

        st = {}
        for g in range(N):
            c, b = g // B, g % B
            p = g % 4
            ld[g].wait()
            if b == 0:
                pe_pend[c].wait()
            pv = peb[c % 2]
            xv = xb[p]

            def add_row(r, carry):
                for j in range(GROUPS):
                    s = (r, pl.ds(j * _LANES, _LANES))
                    xv[s] = xv[s] + pv[s]
                return carry

            lax.fori_loop(0, ROWS, add_row, 0)

            st[g] = pltpu.async_copy(
                xv, o_hbm.at[b, pl.ds(l0(c), ROWS)], ssem[p])

            if b == B - 1 and c + 2 < n_sub:
                pe_pend[c + 2] = pltpu.async_copy(
                    pe_hbm.at[pl.ds(l0(c + 2), ROWS)], peb[c % 2], psem[c % 2])

            h = g + 3
            if h < N:
                if g >= 1:
                    st[g - 1].wait()
                hc, hb = h // B, h % B
                ld[h] = pltpu.async_copy(
                    x_hbm.at[hb, pl.ds(l0(hc), ROWS)], xb[h % 4], lsem[h % 4])

        for g in range(max(0, N - 4), N):
            st[g].wait()

    return k


def kernel(x, pe):
    B, L, D = x.shape
    return _make_sc_add(B, L, D)(x, pe[:L])


# speedup vs baseline: 1.1384x; 1.1384x over previous
import functools
import jax
import jax.numpy as jnp
from jax import lax
from jax.experimental import pallas as pl
from jax.experimental.pallas import tpu as pltpu
from jax.experimental.pallas import tpu_sc as plsc

_NC = 2
_NS = 16
_NW = _NC * _NS
_LANES = 16


def _make_sc_add(B, L, D):
    ROWS = 8
    l_per_w = L // _NW
    n_sub = l_per_w // ROWS
    GROUPS = D // _LANES
    RING = 3

    mesh = plsc.VectorSubcoreMesh(core_axis_name="c", subcore_axis_name="s")

    @functools.partial(
        pl.kernel,
        mesh=mesh,
        out_type=jax.ShapeDtypeStruct((B, L, D), jnp.float32),
        scratch_types=(
            [pltpu.VMEM((ROWS, D), jnp.float32) for _ in range(RING * B)]
            + [pltpu.VMEM((ROWS, D), jnp.float32) for _ in range(2)]
            + [pltpu.SemaphoreType.DMA for _ in range(RING)]
            + [pltpu.SemaphoreType.DMA for _ in range(RING)]
            + [pltpu.SemaphoreType.DMA for _ in range(2)]
        ),
    )
    def k(x_hbm, pe_hbm, o_hbm, *refs):
        xb = refs[0:RING * B]
        peb = refs[RING * B:RING * B + 2]
        lsem = refs[RING * B + 2:RING * B + 2 + RING]
        ssem = refs[RING * B + 2 + RING:RING * B + 2 + 2 * RING]
        psem = refs[RING * B + 2 + 2 * RING:]

        wid = lax.axis_index("s") * _NC + lax.axis_index("c")
        base_l = wid * l_per_w

        def l0(t):
            return base_l + t * ROWS

        def load_chunk(t):
            q = t % RING
            return [
                pltpu.async_copy(
                    x_hbm.at[b, pl.ds(l0(t), ROWS)], xb[q * B + b], lsem[q])
                for b in range(B)
            ]

        pe_pend = {}
        for t in range(min(2, n_sub)):
            pe_pend[t] = pltpu.async_copy(
                pe_hbm.at[pl.ds(l0(t), ROWS)], peb[t % 2], psem[t % 2])
        ld = {}
        for t in range(min(2, n_sub)):
            ld[t] = load_chunk(t)

        st = {}
        for t in range(n_sub):
            q = t % RING
            for h in ld[t]:
                h.wait()
            pe_pend[t].wait()
            pv = peb[t % 2]
            xset = [xb[q * B + b] for b in range(B)]

            def add_col(j, carry):
                col = pl.ds(j * _LANES, _LANES)
                for r in range(ROWS):
                    pvreg = pv[r, col]
                    for b in range(B):
                        xv = xset[b]
                        xv[r, col] = xv[r, col] + pvreg
                return carry

            lax.fori_loop(0, GROUPS, add_col, 0)

            st[t] = [
                pltpu.async_copy(
                    xset[b], o_hbm.at[b, pl.ds(l0(t), ROWS)], ssem[q])
                for b in range(B)
            ]

            if t + 2 < n_sub:
                pe_pend[t + 2] = pltpu.async_copy(
                    pe_hbm.at[pl.ds(l0(t + 2), ROWS)], peb[t % 2], psem[t % 2])
                if t >= 1:
                    for h in st[t - 1]:
                        h.wait()
                ld[t + 2] = load_chunk(t + 2)

        for t in range(max(0, n_sub - 3), n_sub):
            for h in st[t]:
                h.wait()

    return k


def kernel(x, pe):
    B, L, D = x.shape
    return _make_sc_add(B, L, D)(x, pe[:L])
